# Initial kernel scaffold; baseline (speedup 1.0000x reference)
import jax
import jax.numpy as jnp
from jax.experimental import pallas as pl


def kernel(x, edge_index, W1, b1, W2, b2, Wc, bc):
    raise NotImplementedError("write your pallas kernel here")


# speedup vs baseline: 8.2565x; 8.2565x over previous
import functools

import jax
import jax.numpy as jnp
from jax import lax
from jax.experimental import pallas as pl
from jax.experimental.pallas import tpu as pltpu
from jax.experimental.pallas import tpu_sc as plsc

N_NODES = 10000
IN_DIM = 128
HID_DIM = 128
OUT_DIM = 64

NC, NS = 2, 16
NW = NC * NS
CHUNK = 128
T_CHUNKS = 80
E_PAD = NW * T_CHUNKS * CHUNK
ROWS_PAD = 10240
RPT = ROWS_PAD // NS
DEG_W = 8

_mesh = plsc.VectorSubcoreMesh(
    core_axis_name="c", subcore_axis_name="s", num_cores=NC, num_subcores=NS)


def _fill(buf, nrows, ncols, value):
    v = jnp.full((16,), value, jnp.float32)

    def row(i, carry):
        for j in range(ncols // 16):
            buf[i, pl.ds(j * 16, 16)] = v
        return carry

    lax.fori_loop(0, nrows, row, 0)


@functools.partial(
    pl.kernel,
    out_type=jax.ShapeDtypeStruct((NC * ROWS_PAD, DEG_W), jnp.float32),
    mesh=_mesh,
    scratch_types=[
        pltpu.VMEM((T_CHUNKS, CHUNK), jnp.int32),
        pltpu.VMEM((CHUNK, DEG_W), jnp.float32),
        pltpu.VMEM((CHUNK, DEG_W), jnp.float32),
        pltpu.VMEM_SHARED((ROWS_PAD, DEG_W), jnp.float32),
    ],
)
def _deg_kernel(dst_hbm, out_hbm, idx_v, ones_v, zero_v, acc_sh):
    c = lax.axis_index("c")
    s = lax.axis_index("s")
    wid = c * NS + s
    pltpu.sync_copy(dst_hbm.at[wid], idx_v)
    _fill(ones_v, CHUNK, DEG_W, 1.0)
    _fill(zero_v, CHUNK, DEG_W, 0.0)

    base = s * RPT
    for k in range(RPT // CHUNK):
        pltpu.sync_copy(zero_v, acc_sh.at[pl.ds(base + k * CHUNK, CHUNK)])
    plsc.subcore_barrier()

    def body(t, carry):
        pltpu.sync_copy(ones_v, acc_sh.at[idx_v.at[t]], add=True)
        return carry

    lax.fori_loop(0, T_CHUNKS, body, 0)
    plsc.subcore_barrier()
    pltpu.sync_copy(acc_sh.at[pl.ds(base, RPT)],
                    out_hbm.at[pl.ds(c * ROWS_PAD + base, RPT)])


@functools.partial(
    pl.kernel,
    out_type=jax.ShapeDtypeStruct((NC * ROWS_PAD, HID_DIM), jnp.float32),
    mesh=_mesh,
    scratch_types=[
        pltpu.VMEM((2, 2, CHUNK), jnp.int32),
        pltpu.VMEM((CHUNK, HID_DIM), jnp.float32),
        pltpu.VMEM((CHUNK, HID_DIM), jnp.float32),
        pltpu.SemaphoreType.DMA,
        pltpu.SemaphoreType.DMA,
        pltpu.SemaphoreType.DMA,
        pltpu.SemaphoreType.DMA,
        pltpu.VMEM_SHARED((ROWS_PAD, HID_DIM), jnp.float32),
    ],
)
def _segsum_kernel(g_hbm, il_hbm, out_hbm,
                   ei_v, buf0, buf1, semg0, semg1, semi0, semi1, acc_sh):
    c = lax.axis_index("c")
    s = lax.axis_index("s")
    wid = c * NS + s
    cid0 = wid * T_CHUNKS

    _fill(buf0, CHUNK, HID_DIM, 0.0)
    base = s * RPT
    for k in range(RPT // CHUNK):
        pltpu.sync_copy(buf0, acc_sh.at[pl.ds(base + k * CHUNK, CHUNK)])
    plsc.subcore_barrier()

    bufs = (buf0, buf1)
    semg = (semg0, semg1)
    semi = (semi0, semi1)

    pltpu.async_copy(il_hbm.at[cid0], ei_v.at[0], semi0).wait()
    pltpu.async_copy(g_hbm.at[ei_v.at[0, 0]], buf0, semg0)
    pltpu.async_copy(il_hbm.at[cid0 + 1], ei_v.at[1], semi1)

    def body(i, carry):
        for b in range(2):
            t = 2 * i + b

            @pl.when(t + 1 < T_CHUNKS)
            def _():
                pltpu.make_async_copy(il_hbm.at[cid0 + t + 1],
                                      ei_v.at[1 - b], semi[1 - b]).wait()
                pltpu.async_copy(g_hbm.at[ei_v.at[1 - b, 0]],
                                 bufs[1 - b], semg[1 - b])

            pltpu.make_async_copy(g_hbm.at[ei_v.at[b, 0]],
                                  bufs[b], semg[b]).wait()
            pltpu.sync_copy(bufs[b], acc_sh.at[ei_v.at[b, 1]], add=True)

            @pl.when(t + 2 < T_CHUNKS)
            def _():
                pltpu.async_copy(il_hbm.at[cid0 + t + 2], ei_v.at[b], semi[b])

        return carry

    lax.fori_loop(0, T_CHUNKS // 2, body, 0)
    plsc.subcore_barrier()
    pltpu.sync_copy(acc_sh.at[pl.ds(base, RPT)],
                    out_hbm.at[pl.ds(c * ROWS_PAD + base, RPT)])


R = 1024
GRID = ROWS_PAD // R


def _pre_body(deg0, deg1, x, w, g, dinv_o):
    deg = deg0[...] + deg1[...] + 1.0
    dinv = lax.rsqrt(deg)
    h = jnp.dot(x[...], w[...], preferred_element_type=jnp.float32)
    g[...] = h * dinv[:, None]
    dinv_o[...] = dinv


def _mid_body(s0, s1, g1, dinv, b1, w2, g2):
    d = dinv[...]
    o = (s0[...] + s1[...] + g1[...]) * d[:, None] + b1[...][None, :]
    h = jnp.maximum(o, 0.0)
    g2[...] = jnp.dot(h, w2[...], preferred_element_type=jnp.float32) * d[:, None]


def _fin_body(s0, s1, g2, dinv, b2, wc, bc, out):
    d = dinv[...]
    h = (s0[...] + s1[...] + g2[...]) * d[:, None] + b2[...][None, :]
    out[...] = jnp.dot(h, wc[...], preferred_element_type=jnp.float32) + bc[...][None, :]


def _vec_spec():
    return pl.BlockSpec((R,), lambda i: (i,))


def _mat_spec(ncols):
    return pl.BlockSpec((R, ncols), lambda i: (i, 0))


def _full_spec(shape):
    nd = len(shape)
    return pl.BlockSpec(shape, lambda i: (0,) * nd)


def kernel(x, edge_index, W1, b1, W2, b2, Wc, bc):
    ei = edge_index.astype(jnp.int32)
    e = ei.shape[1]
    pad = E_PAD - e
    src_p = jnp.concatenate([ei[0], jnp.zeros((pad,), jnp.int32)])
    dst_p = jnp.concatenate([ei[1], jnp.full((pad,), N_NODES, jnp.int32)])
    dst3 = dst_p.reshape(NW, T_CHUNKS, CHUNK)
    il3 = jnp.concatenate(
        [src_p.reshape(NW * T_CHUNKS, 1, CHUNK),
         dst_p.reshape(NW * T_CHUNKS, 1, CHUNK)], axis=1)

    xp = jnp.concatenate(
        [x, jnp.zeros((ROWS_PAD - N_NODES, IN_DIM), x.dtype)])

    degp = _deg_kernel(dst3)
    deg0 = degp[:ROWS_PAD, 0]
    deg1 = degp[ROWS_PAD:, 0]

    g1, dinv = pl.pallas_call(
        _pre_body,
        grid=(GRID,),
        in_specs=[_vec_spec(), _vec_spec(), _mat_spec(IN_DIM),
                  _full_spec((IN_DIM, HID_DIM))],
        out_specs=[_mat_spec(HID_DIM), _vec_spec()],
        out_shape=[jax.ShapeDtypeStruct((ROWS_PAD, HID_DIM), jnp.float32),
                   jax.ShapeDtypeStruct((ROWS_PAD,), jnp.float32)],
    )(deg0, deg1, xp, W1)

    s1p = _segsum_kernel(g1, il3)
    g2 = pl.pallas_call(
        _mid_body,
        grid=(GRID,),
        in_specs=[_mat_spec(HID_DIM), _mat_spec(HID_DIM), _mat_spec(HID_DIM),
                  _vec_spec(), _full_spec((HID_DIM,)),
                  _full_spec((HID_DIM, HID_DIM))],
        out_specs=_mat_spec(HID_DIM),
        out_shape=jax.ShapeDtypeStruct((ROWS_PAD, HID_DIM), jnp.float32),
    )(s1p[:ROWS_PAD], s1p[ROWS_PAD:], g1, dinv, b1, W2)

    s2p = _segsum_kernel(g2, il3)
    out = pl.pallas_call(
        _fin_body,
        grid=(GRID,),
        in_specs=[_mat_spec(HID_DIM), _mat_spec(HID_DIM), _mat_spec(HID_DIM),
                  _vec_spec(), _full_spec((HID_DIM,)),
                  _full_spec((HID_DIM, OUT_DIM)), _full_spec((OUT_DIM,))],
        out_specs=pl.BlockSpec((R, OUT_DIM), lambda i: (i, 0)),
        out_shape=jax.ShapeDtypeStruct((ROWS_PAD, OUT_DIM), jnp.float32),
    )(s2p[:ROWS_PAD], s2p[ROWS_PAD:], g2, dinv, b2, Wc, bc)
    return out[:N_NODES]


# speedup vs baseline: 10.8152x; 1.3099x over previous
import functools

import jax
import jax.numpy as jnp
from jax import lax
from jax.experimental import pallas as pl
from jax.experimental.pallas import tpu as pltpu
from jax.experimental.pallas import tpu_sc as plsc

N_NODES = 10000
IN_DIM = 128
HID_DIM = 128
OUT_DIM = 64

NC, NS = 2, 16
NW = NC * NS
CHUNK = 128
T_CHUNKS = 80
E_PAD = NW * T_CHUNKS * CHUNK
ROWS_PAD = 10240
RPT = ROWS_PAD // NS
DEG_W = 8

_mesh = plsc.VectorSubcoreMesh(
    core_axis_name="c", subcore_axis_name="s", num_cores=NC, num_subcores=NS)


def _fill(buf, nrows, ncols, value):
    v = jnp.full((16,), value, jnp.float32)

    def row(i, carry):
        for j in range(ncols // 16):
            buf[i, pl.ds(j * 16, 16)] = v
        return carry

    lax.fori_loop(0, nrows, row, 0)


@functools.partial(
    pl.kernel,
    out_type=jax.ShapeDtypeStruct((NC * ROWS_PAD, DEG_W), jnp.float32),
    mesh=_mesh,
    scratch_types=[
        pltpu.VMEM((T_CHUNKS, CHUNK), jnp.int32),
        pltpu.VMEM((CHUNK, DEG_W), jnp.float32),
        pltpu.VMEM((CHUNK, DEG_W), jnp.float32),
        pltpu.VMEM_SHARED((ROWS_PAD, DEG_W), jnp.float32),
    ],
)
def _deg_kernel(dst_hbm, out_hbm, idx_v, ones_v, zero_v, acc_sh):
    c = lax.axis_index("c")
    s = lax.axis_index("s")
    wid = c * NS + s
    pltpu.sync_copy(dst_hbm.at[wid], idx_v)
    _fill(ones_v, CHUNK, DEG_W, 1.0)
    _fill(zero_v, CHUNK, DEG_W, 0.0)

    base = s * RPT
    for k in range(RPT // CHUNK):
        pltpu.sync_copy(zero_v, acc_sh.at[pl.ds(base + k * CHUNK, CHUNK)])
    plsc.subcore_barrier()

    def body(t, carry):
        pltpu.sync_copy(ones_v, acc_sh.at[idx_v.at[t]], add=True)
        return carry

    lax.fori_loop(0, T_CHUNKS, body, 0)
    plsc.subcore_barrier()
    pltpu.sync_copy(acc_sh.at[pl.ds(base, RPT)],
                    out_hbm.at[pl.ds(c * ROWS_PAD + base, RPT)])


@functools.partial(
    pl.kernel,
    out_type=jax.ShapeDtypeStruct((NC * ROWS_PAD, HID_DIM), jnp.float32),
    mesh=_mesh,
    scratch_types=[
        pltpu.VMEM((2, 2, CHUNK), jnp.int32),
        pltpu.VMEM((CHUNK, HID_DIM), jnp.float32),
        pltpu.VMEM((CHUNK, HID_DIM), jnp.float32),
        pltpu.SemaphoreType.DMA,
        pltpu.SemaphoreType.DMA,
        pltpu.SemaphoreType.DMA,
        pltpu.SemaphoreType.DMA,
        pltpu.VMEM_SHARED((ROWS_PAD, HID_DIM), jnp.float32),
    ],
)
def _segsum_kernel(g_hbm, il_hbm, out_hbm,
                   ei_v, buf0, buf1, semg0, semg1, semi0, semi1, acc_sh):
    c = lax.axis_index("c")
    s = lax.axis_index("s")
    wid = c * NS + s
    cid0 = wid * T_CHUNKS

    _fill(buf0, CHUNK, HID_DIM, 0.0)
    base = s * RPT
    for k in range(RPT // CHUNK):
        pltpu.sync_copy(buf0, acc_sh.at[pl.ds(base + k * CHUNK, CHUNK)])
    plsc.subcore_barrier()

    bufs = (buf0, buf1)
    semg = (semg0, semg1)
    semi = (semi0, semi1)

    pltpu.async_copy(il_hbm.at[cid0], ei_v.at[0], semi0).wait()
    pltpu.async_copy(g_hbm.at[ei_v.at[0, 0]], buf0, semg0)
    pltpu.async_copy(il_hbm.at[cid0 + 1], ei_v.at[1], semi1)

    def body(i, carry):
        for b in range(2):
            t = 2 * i + b

            @pl.when(t + 1 < T_CHUNKS)
            def _():
                pltpu.make_async_copy(il_hbm.at[cid0 + t + 1],
                                      ei_v.at[1 - b], semi[1 - b]).wait()
                pltpu.async_copy(g_hbm.at[ei_v.at[1 - b, 0]],
                                 bufs[1 - b], semg[1 - b])

            pltpu.make_async_copy(g_hbm.at[ei_v.at[b, 0]],
                                  bufs[b], semg[b]).wait()
            pltpu.sync_copy(bufs[b], acc_sh.at[ei_v.at[b, 1]], add=True)

            @pl.when(t + 2 < T_CHUNKS)
            def _():
                pltpu.async_copy(il_hbm.at[cid0 + t + 2], ei_v.at[b], semi[b])

        return carry

    lax.fori_loop(0, T_CHUNKS // 2, body, 0)
    plsc.subcore_barrier()
    pltpu.sync_copy(acc_sh.at[pl.ds(base, RPT)],
                    out_hbm.at[pl.ds(c * ROWS_PAD + base, RPT)])


R = 1024
GRID = ROWS_PAD // R


def _pre_body(deg0, deg1, x, w, g, dinv_o):
    deg = deg0[...] + deg1[...] + 1.0
    dinv = lax.rsqrt(deg)
    h = jnp.dot(x[...], w[...], preferred_element_type=jnp.float32)
    g[...] = h * dinv[:, None]
    dinv_o[...] = dinv


def _mid_body(s0, s1, g1, dinv, b1, w2, g2):
    d = dinv[...]
    o = (s0[...] + s1[...] + g1[...]) * d[:, None] + b1[...][None, :]
    h = jnp.maximum(o, 0.0)
    g2[...] = jnp.dot(h, w2[...], preferred_element_type=jnp.float32) * d[:, None]


def _fin_body(s0, s1, g2, dinv, b2, wc, bc, out):
    d = dinv[...]
    h = (s0[...] + s1[...] + g2[...]) * d[:, None] + b2[...][None, :]
    out[...] = jnp.dot(h, wc[...], preferred_element_type=jnp.float32) + bc[...][None, :]


def _vec_spec():
    return pl.BlockSpec((R,), lambda i: (i,))


def _mat_spec(ncols):
    return pl.BlockSpec((R, ncols), lambda i: (i, 0))


def _full_spec(shape):
    nd = len(shape)
    return pl.BlockSpec(shape, lambda i: (0,) * nd)


def kernel(x, edge_index, W1, b1, W2, b2, Wc, bc):
    ei = edge_index.astype(jnp.int32)
    e = ei.shape[1]
    ppw = (E_PAD - e) // NW
    src_p = jnp.concatenate(
        [ei[0].reshape(NW, e // NW),
         jnp.zeros((NW, ppw), jnp.int32)], axis=1)
    dst_p = jnp.concatenate(
        [ei[1].reshape(NW, e // NW),
         jnp.broadcast_to(N_NODES + jnp.arange(ppw, dtype=jnp.int32),
                          (NW, ppw))], axis=1)
    dst3 = dst_p.reshape(NW, T_CHUNKS, CHUNK)
    il3 = jnp.concatenate(
        [src_p.reshape(NW * T_CHUNKS, 1, CHUNK),
         dst_p.reshape(NW * T_CHUNKS, 1, CHUNK)], axis=1)

    xp = jnp.concatenate(
        [x, jnp.zeros((ROWS_PAD - N_NODES, IN_DIM), x.dtype)])

    degp = _deg_kernel(dst3)
    deg0 = degp[:ROWS_PAD, 0]
    deg1 = degp[ROWS_PAD:, 0]

    g1, dinv = pl.pallas_call(
        _pre_body,
        grid=(GRID,),
        in_specs=[_vec_spec(), _vec_spec(), _mat_spec(IN_DIM),
                  _full_spec((IN_DIM, HID_DIM))],
        out_specs=[_mat_spec(HID_DIM), _vec_spec()],
        out_shape=[jax.ShapeDtypeStruct((ROWS_PAD, HID_DIM), jnp.float32),
                   jax.ShapeDtypeStruct((ROWS_PAD,), jnp.float32)],
    )(deg0, deg1, xp, W1)

    s1p = _segsum_kernel(g1, il3)
    g2 = pl.pallas_call(
        _mid_body,
        grid=(GRID,),
        in_specs=[_mat_spec(HID_DIM), _mat_spec(HID_DIM), _mat_spec(HID_DIM),
                  _vec_spec(), _full_spec((HID_DIM,)),
                  _full_spec((HID_DIM, HID_DIM))],
        out_specs=_mat_spec(HID_DIM),
        out_shape=jax.ShapeDtypeStruct((ROWS_PAD, HID_DIM), jnp.float32),
    )(s1p[:ROWS_PAD], s1p[ROWS_PAD:], g1, dinv, b1, W2)

    s2p = _segsum_kernel(g2, il3)
    out = pl.pallas_call(
        _fin_body,
        grid=(GRID,),
        in_specs=[_mat_spec(HID_DIM), _mat_spec(HID_DIM), _mat_spec(HID_DIM),
                  _vec_spec(), _full_spec((HID_DIM,)),
                  _full_spec((HID_DIM, OUT_DIM)), _full_spec((OUT_DIM,))],
        out_specs=pl.BlockSpec((R, OUT_DIM), lambda i: (i, 0)),
        out_shape=jax.ShapeDtypeStruct((ROWS_PAD, OUT_DIM), jnp.float32),
    )(s2p[:ROWS_PAD], s2p[ROWS_PAD:], g2, dinv, b2, Wc, bc)
    return out[:N_NODES]
